# Initial kernel scaffold; baseline (speedup 1.0000x reference)
import jax
import jax.numpy as jnp
from jax.experimental import pallas as pl


def kernel(x_cat, tables):
    raise NotImplementedError("write your pallas kernel here")


# speedup vs baseline: 1.0323x; 1.0323x over previous
import functools

import jax
import jax.numpy as jnp
import numpy as np
from jax import lax
from jax.experimental import pallas as pl
from jax.experimental.pallas import tpu as pltpu
from jax.experimental.pallas import tpu_sc as plsc

N_FIELDS = 26
VOCAB = 100000
D_TOKEN = 64
BATCH = 16384

NC = 2
NS = 16
L = 16
NW = NC * NS
ROWS_PER_W = BATCH * N_FIELDS // NW

CB = 64
ROWS_PER_CHUNK = CB * N_FIELDS
N_CHUNKS = ROWS_PER_W // ROWS_PER_CHUNK
G = 128
N_G = ROWS_PER_CHUNK // G

_OFFS = np.tile(
    (np.arange(N_FIELDS, dtype=np.int32) * VOCAB), ROWS_PER_CHUNK // N_FIELDS
)


def _body(xcat_hbm, table_hbm, offs_hbm, out_hbm, xi_v, idx_v, rows_v,
          offs_v, sem):
    wid = lax.axis_index("s") * NC + lax.axis_index("c")
    base = wid * ROWS_PER_W
    pltpu.sync_copy(offs_hbm, offs_v)

    def chunk(c, carry):
        r0 = base + c * ROWS_PER_CHUNK
        pltpu.sync_copy(xcat_hbm.at[pl.ds(r0, ROWS_PER_CHUNK)], xi_v)
        for g in range(N_G):
            for j in range(G // L):
                sl = pl.ds(g * G + j * L, L)
                idx_v[g, pl.ds(j * L, L)] = xi_v[sl] + offs_v[sl]
        cps = [
            pltpu.async_copy(
                table_hbm.at[idx_v.at[g]],
                rows_v.at[pl.ds(g * G, G)],
                sem,
            )
            for g in range(N_G)
        ]
        for cp in cps:
            cp.wait()
        pltpu.sync_copy(rows_v, out_hbm.at[pl.ds(r0, ROWS_PER_CHUNK)])
        return carry

    lax.fori_loop(0, N_CHUNKS, chunk, 0)


@jax.jit
def _lookup(xcat_flat, table_flat, offs):
    mesh = plsc.VectorSubcoreMesh(core_axis_name="c", subcore_axis_name="s")
    return pl.kernel(
        _body,
        mesh=mesh,
        out_type=jax.ShapeDtypeStruct((BATCH * N_FIELDS, D_TOKEN),
                                      jnp.float32),
        scratch_types=[
            pltpu.VMEM((ROWS_PER_CHUNK,), jnp.int32),
            pltpu.VMEM((N_G, G), jnp.int32),
            pltpu.VMEM((ROWS_PER_CHUNK, D_TOKEN), jnp.float32),
            pltpu.VMEM((ROWS_PER_CHUNK,), jnp.int32),
            pltpu.SemaphoreType.DMA,
        ],
        compiler_params=pltpu.CompilerParams(use_tc_tiling_on_sc=False),
    )(xcat_flat, table_flat, offs)


def kernel(x_cat, tables):
    xcat_flat = x_cat.astype(jnp.int32).reshape(BATCH * N_FIELDS)
    table_flat = tables.reshape(N_FIELDS * VOCAB, D_TOKEN)
    offs = jnp.asarray(_OFFS)
    out = _lookup(xcat_flat, table_flat, offs)
    return out.reshape(BATCH, N_FIELDS, D_TOKEN)


# speedup vs baseline: 2.4158x; 2.3403x over previous
import functools

import jax
import jax.numpy as jnp
import numpy as np
from jax import lax
from jax.experimental import pallas as pl
from jax.experimental.pallas import tpu as pltpu
from jax.experimental.pallas import tpu_sc as plsc

N_FIELDS = 26
VOCAB = 100000
D_TOKEN = 64
BATCH = 16384

NC = 2
NS = 16
L = 16
NW = NC * NS
N_ROWS = N_FIELDS * D_TOKEN
ROWS_PER_W = N_ROWS // NW
OC = 8192
N_OC = BATCH // OC
GATHER_ITERS = OC // L


def _body(xc_hbm, tab_hbm, out_hbm, tab_v, idx_v, out_v, sem):
    wid = lax.axis_index("s") * NC + lax.axis_index("c")
    r0 = wid * ROWS_PER_W

    def row_step(i, last_f):
        r = r0 + i
        f = r >> 6
        d = r & (D_TOKEN - 1)

        @pl.when(f != last_f)
        def _():
            pltpu.sync_copy(xc_hbm.at[f], idx_v)

        pltpu.sync_copy(tab_hbm.at[f, d], tab_v)

        def chunk(c, carry):
            def gather16(j, carry2):
                sl = pl.ds(c * OC + j * L, L)
                idx16 = idx_v[sl]
                out_v[pl.ds(j * L, L)] = plsc.load_gather(tab_v, [idx16])
                return carry2

            lax.fori_loop(0, GATHER_ITERS, gather16, 0)
            pltpu.sync_copy(out_v, out_hbm.at[f, d, pl.ds(c * OC, OC)])
            return carry

        lax.fori_loop(0, N_OC, chunk, 0)
        return f

    lax.fori_loop(0, ROWS_PER_W, row_step, -1)


@jax.jit
def _lookup(xc_t, tab_t):
    mesh = plsc.VectorSubcoreMesh(core_axis_name="c", subcore_axis_name="s")
    return pl.kernel(
        _body,
        mesh=mesh,
        out_type=jax.ShapeDtypeStruct((N_FIELDS, D_TOKEN, BATCH),
                                      jnp.float32),
        scratch_types=[
            pltpu.VMEM((VOCAB,), jnp.float32),
            pltpu.VMEM((BATCH,), jnp.int32),
            pltpu.VMEM((OC,), jnp.float32),
            pltpu.SemaphoreType.DMA,
        ],
        compiler_params=pltpu.CompilerParams(needs_layout_passes=False),
    )(xc_t, tab_t)


def kernel(x_cat, tables):
    xc_t = x_cat.astype(jnp.int32).T
    tab_t = jnp.transpose(tables, (0, 2, 1))
    out_t = _lookup(xc_t, tab_t)
    return jnp.transpose(out_t, (2, 0, 1))


# speedup vs baseline: 2.4879x; 1.0298x over previous
import functools

import jax
import jax.numpy as jnp
import numpy as np
from jax import lax
from jax.experimental import pallas as pl
from jax.experimental.pallas import tpu as pltpu
from jax.experimental.pallas import tpu_sc as plsc

N_FIELDS = 26
VOCAB = 100000
D_TOKEN = 64
BATCH = 16384

NC = 2
NS = 16
L = 16
NW = NC * NS
N_ROWS = N_FIELDS * D_TOKEN
ROWS_PER_W = N_ROWS // NW
OC = 4096
N_OC = BATCH // OC
UNROLL = 8
GATHER_ITERS = OC // (L * UNROLL)


def _body(xc_hbm, tab_hbm, out_hbm, tab_v, idx_v, out_v, sems):
    wid = lax.axis_index("s") * NC + lax.axis_index("c")
    r0 = wid * ROWS_PER_W

    def row_step(i, last_f):
        r = r0 + i
        f = r >> 6
        d = r & (D_TOKEN - 1)

        @pl.when(f != last_f)
        def _():
            pltpu.sync_copy(xc_hbm.at[f], idx_v)

        pltpu.sync_copy(tab_hbm.at[f, d], tab_v)

        for c in range(N_OC):
            buf = c & 1

            if c >= 2:
                pltpu.make_async_copy(
                    out_v.at[buf], out_hbm.at[f, d, pl.ds(0, OC)],
                    sems.at[buf]).wait()

            def gather16(j, carry2, c=c, buf=buf):
                base = j * (L * UNROLL)
                for u in range(UNROLL):
                    sl_in = pl.ds(c * OC + base + u * L, L)
                    sl_out = pl.ds(base + u * L, L)
                    out_v[buf, sl_out] = plsc.load_gather(
                        tab_v, [idx_v[sl_in]])
                return carry2

            lax.fori_loop(0, GATHER_ITERS, gather16, 0)
            pltpu.async_copy(
                out_v.at[buf], out_hbm.at[f, d, pl.ds(c * OC, OC)],
                sems.at[buf])
        for buf in range(2):
            pltpu.make_async_copy(
                out_v.at[buf], out_hbm.at[f, d, pl.ds(0, OC)],
                sems.at[buf]).wait()
        return f

    lax.fori_loop(0, ROWS_PER_W, row_step, -1)


@jax.jit
def _lookup(xc_t, tab_t):
    mesh = plsc.VectorSubcoreMesh(core_axis_name="c", subcore_axis_name="s")
    return pl.kernel(
        _body,
        mesh=mesh,
        out_type=jax.ShapeDtypeStruct((N_FIELDS, D_TOKEN, BATCH),
                                      jnp.float32),
        scratch_types=[
            pltpu.VMEM((VOCAB,), jnp.float32),
            pltpu.VMEM((BATCH,), jnp.int32),
            pltpu.VMEM((2, OC), jnp.float32),
            pltpu.SemaphoreType.DMA((2,)),
        ],
        compiler_params=pltpu.CompilerParams(needs_layout_passes=False),
    )(xc_t, tab_t)


def kernel(x_cat, tables):
    xc_t = x_cat.astype(jnp.int32).T
    tab_t = jnp.transpose(tables, (0, 2, 1))
    out_t = _lookup(xc_t, tab_t)
    return jnp.transpose(out_t, (2, 0, 1))


# speedup vs baseline: 2.4883x; 1.0001x over previous
import functools

import jax
import jax.numpy as jnp
import numpy as np
from jax import lax
from jax.experimental import pallas as pl
from jax.experimental.pallas import tpu as pltpu
from jax.experimental.pallas import tpu_sc as plsc

N_FIELDS = 26
VOCAB = 100000
D_TOKEN = 64
BATCH = 16384

NC = 2
NS = 16
L = 16
NW = NC * NS
N_ROWS = N_FIELDS * D_TOKEN
ROWS_PER_W = N_ROWS // NW

TSPLIT = 50048
HB = BATCH // 2
SC_CHUNK = 4096
POS_BITS = 13
POS_MASK = HB - 1


def _body(xc_hbm, tab_hbm, out_hbm, bufA, bufB, combo, out_v, stage, ns_s,
          semA, semB):
    wid = lax.axis_index("s") * NC + lax.axis_index("c")
    r0 = wid * ROWS_PER_W
    iota16 = lax.iota(jnp.int32, L)

    def issue_tab(rr):
        fr = rr >> 6
        dr = rr & (D_TOKEN - 1)
        pltpu.async_copy(tab_hbm.at[fr, dr, pl.ds(0, TSPLIT)], bufA, semA)
        pltpu.async_copy(tab_hbm.at[fr, dr, pl.ds(TSPLIT, VOCAB - TSPLIT)],
                         bufB, semB)

    issue_tab(r0)

    def row_step(i, last_f):
        r = r0 + i
        f = r >> 6
        d = r & (D_TOKEN - 1)

        @pl.when(f != last_f)
        def _():
            for h in range(2):
                base = HB * h

                def cchunk(q, ptrs, base=base):
                    off = pl.multiple_of(base + SC_CHUNK * q, 128)
                    pltpu.sync_copy(
                        xc_hbm.at[f, pl.ds(off, SC_CHUNK)],
                        stage)

                    def citer(t, ptrs2, q=q):
                        pA, pB = ptrs2
                        w = stage[pl.ds(t * L, L)]
                        pos = t * L + SC_CHUNK * q + iota16
                        mA = w < TSPLIT
                        miA = mA.astype(jnp.int32)
                        csA = plsc.cumsum(miA)
                        wsh = w << POS_BITS
                        plsc.store_scatter(combo, [pA - 1 + csA], wsh | pos,
                                           mask=mA)
                        totA = jnp.sum(miA)
                        csB = plsc.cumsum(1 - miA)
                        plsc.store_scatter(
                            combo, [pB - csB],
                            (wsh - (TSPLIT << POS_BITS)) | pos,
                            mask=jnp.logical_not(mA))
                        return (pA + totA, pB - (L - totA))

                    return lax.fori_loop(0, SC_CHUNK // L, citer, ptrs)

                ptrA_f, ptrB_f = lax.fori_loop(
                    0, HB // SC_CHUNK, cchunk,
                    (jnp.int32(base), jnp.int32(base + HB)))
                ns_s[2 * h] = ptrA_f - base
                ns_s[2 * h + 1] = base + HB - ptrB_f

        pltpu.make_async_copy(
            tab_hbm.at[f, d, pl.ds(0, TSPLIT)], bufA, semA).wait()
        pltpu.make_async_copy(
            tab_hbm.at[f, d, pl.ds(TSPLIT, VOCAB - TSPLIT)], bufB,
            semB).wait()

        for h in range(2):
            base = HB * h
            nA = ns_s[2 * h]
            nB = ns_s[2 * h + 1]

            boundA = base + nA
            kA = (nA + L - 1) >> 4

            def aiter(j, c2, base=base, boundA=boundA):
                off = base + j * L
                w = combo[pl.ds(off, L)]
                msk = (off + iota16) < boundA
                vals = plsc.load_gather(bufA, [w >> POS_BITS], mask=msk)
                plsc.store_scatter(out_v, [w & POS_MASK], vals, mask=msk)
                return c2

            lax.fori_loop(0, kA, aiter, 0)

            if h == 1:
                @pl.when(i + 1 < ROWS_PER_W)
                def _():
                    rn = r + 1
                    pltpu.async_copy(
                        tab_hbm.at[rn >> 6, rn & (D_TOKEN - 1),
                                   pl.ds(0, TSPLIT)], bufA, semA)

            bstart = base + HB - nB
            bal = bstart & ~(L - 1)
            kB = (base + HB - bal) >> 4

            def biter(j, c2, bal=bal, bstart=bstart):
                off = bal + j * L
                w = combo[pl.ds(off, L)]
                msk = (off + iota16) >= bstart
                vals = plsc.load_gather(bufB, [w >> POS_BITS], mask=msk)
                plsc.store_scatter(out_v, [w & POS_MASK], vals, mask=msk)
                return c2

            lax.fori_loop(0, kB, biter, 0)

            if h == 1:
                @pl.when(i + 1 < ROWS_PER_W)
                def _():
                    rn = r + 1
                    pltpu.async_copy(
                        tab_hbm.at[rn >> 6, rn & (D_TOKEN - 1),
                                   pl.ds(TSPLIT, VOCAB - TSPLIT)], bufB,
                        semB)

            pltpu.sync_copy(out_v, out_hbm.at[f, d, pl.ds(base, HB)])

        return f

    lax.fori_loop(0, ROWS_PER_W, row_step, -1)


@jax.jit
def _lookup(xc_t, tab_t):
    mesh = plsc.VectorSubcoreMesh(core_axis_name="c", subcore_axis_name="s")
    return pl.kernel(
        _body,
        mesh=mesh,
        out_type=jax.ShapeDtypeStruct((N_FIELDS, D_TOKEN, BATCH),
                                      jnp.float32),
        scratch_types=[
            pltpu.VMEM((TSPLIT,), jnp.float32),
            pltpu.VMEM((VOCAB - TSPLIT,), jnp.float32),
            pltpu.VMEM((BATCH,), jnp.int32),
            pltpu.VMEM((HB,), jnp.float32),
            pltpu.VMEM((SC_CHUNK,), jnp.int32),
            pltpu.SMEM((8,), jnp.int32),
            pltpu.SemaphoreType.DMA,
            pltpu.SemaphoreType.DMA,
        ],
        compiler_params=pltpu.CompilerParams(needs_layout_passes=False),
    )(xc_t, tab_t)


def kernel(x_cat, tables):
    xc_t = x_cat.astype(jnp.int32).T
    tab_t = jnp.transpose(tables, (0, 2, 1))
    out_t = _lookup(xc_t, tab_t)
    return jnp.transpose(out_t, (2, 0, 1))


# speedup vs baseline: 2.5039x; 1.0063x over previous
import functools

import jax
import jax.numpy as jnp
import numpy as np
from jax import lax
from jax.experimental import pallas as pl
from jax.experimental.pallas import tpu as pltpu
from jax.experimental.pallas import tpu_sc as plsc

N_FIELDS = 26
VOCAB = 100000
D_TOKEN = 64
BATCH = 16384

NC = 2
NS = 16
L = 16
NW = NC * NS
N_ROWS = N_FIELDS * D_TOKEN
ROWS_PER_W = N_ROWS // NW

TSPLIT = 50048
HB = BATCH // 2
SC_CHUNK = 4096
POS_BITS = 13
POS_MASK = HB - 1


def _body(xc_hbm, tab_hbm, out_hbm, bufA, bufB, combo, out_v, stage, ns_s,
          semA, semB):
    wid = lax.axis_index("s") * NC + lax.axis_index("c")
    r0 = wid * ROWS_PER_W
    iota16 = lax.iota(jnp.int32, L)

    def issue_tab(rr):
        fr = rr >> 6
        dr = rr & (D_TOKEN - 1)
        pltpu.async_copy(tab_hbm.at[fr, dr, pl.ds(0, TSPLIT)], bufA, semA)
        pltpu.async_copy(tab_hbm.at[fr, dr, pl.ds(TSPLIT, VOCAB - TSPLIT)],
                         bufB, semB)

    issue_tab(r0)

    def row_step(i, last_f):
        r = r0 + i
        f = r >> 6
        d = r & (D_TOKEN - 1)

        @pl.when(f != last_f)
        def _():
            for h in range(2):
                base = HB * h

                def cchunk(q, ptrs, base=base):
                    off = pl.multiple_of(base + SC_CHUNK * q, 128)
                    pltpu.sync_copy(
                        xc_hbm.at[f, pl.ds(off, SC_CHUNK)],
                        stage)

                    def citer(t, ptrs2, q=q):
                        pA, pB = ptrs2
                        w = stage[pl.ds(t * L, L)]
                        pos = t * L + SC_CHUNK * q + iota16
                        mA = w < TSPLIT
                        miA = mA.astype(jnp.int32)
                        csA = plsc.cumsum(miA)
                        wsh = w << POS_BITS
                        plsc.store_scatter(combo, [pA - 1 + csA], wsh | pos,
                                           mask=mA)
                        totA = jnp.sum(miA)
                        csB = plsc.cumsum(1 - miA)
                        plsc.store_scatter(
                            combo, [pB - csB],
                            (wsh - (TSPLIT << POS_BITS)) | pos,
                            mask=jnp.logical_not(mA))
                        return (pA + totA, pB - (L - totA))

                    return lax.fori_loop(0, SC_CHUNK // L, citer, ptrs)

                ptrA_f, ptrB_f = lax.fori_loop(
                    0, HB // SC_CHUNK, cchunk,
                    (jnp.int32(base), jnp.int32(base + HB)))
                ns_s[2 * h] = ptrA_f - base
                ns_s[2 * h + 1] = base + HB - ptrB_f

        pltpu.make_async_copy(
            tab_hbm.at[f, d, pl.ds(0, TSPLIT)], bufA, semA).wait()
        pltpu.make_async_copy(
            tab_hbm.at[f, d, pl.ds(TSPLIT, VOCAB - TSPLIT)], bufB,
            semB).wait()

        for h in range(2):
            base = HB * h
            nA = ns_s[2 * h]

            def blk(buf, off):
                w = combo[pl.ds(off, L)]
                vals = plsc.load_gather(buf, [w >> POS_BITS])
                plsc.store_scatter(out_v, [w & POS_MASK], vals)

            kfull = nA >> 4
            k4 = kfull >> 2

            def a4(j, c2, base=base):
                for u in range(4):
                    blk(bufA, base + j * 64 + u * L)
                return c2

            def a1(j, c2, base=base):
                blk(bufA, base + j * L)
                return c2

            lax.fori_loop(0, k4, a4, 0)
            lax.fori_loop(k4 * 4, kfull, a1, 0)

            bnd = base + (nA & ~(L - 1))
            has_bnd = (nA & (L - 1)) > 0

            @pl.when(has_bnd)
            def _(base=base, bnd=bnd, nA=nA):
                w = combo[pl.ds(bnd, L)]
                msk = (bnd + iota16) < base + nA
                vals = plsc.load_gather(bufA, [w >> POS_BITS], mask=msk)
                plsc.store_scatter(out_v, [w & POS_MASK], vals, mask=msk)

            if h == 1:
                @pl.when(i + 1 < ROWS_PER_W)
                def _():
                    rn = r + 1
                    pltpu.async_copy(
                        tab_hbm.at[rn >> 6, rn & (D_TOKEN - 1),
                                   pl.ds(0, TSPLIT)], bufA, semA)

            @pl.when(has_bnd)
            def _(base=base, bnd=bnd, nA=nA):
                w = combo[pl.ds(bnd, L)]
                msk = (bnd + iota16) >= base + nA
                vals = plsc.load_gather(bufB, [w >> POS_BITS], mask=msk)
                plsc.store_scatter(out_v, [w & POS_MASK], vals, mask=msk)

            sB = (nA + L - 1) >> 4
            kB = (HB >> 4) - sB
            kB4 = kB >> 2

            def b4(j, c2, base=base, sB=sB):
                for u in range(4):
                    blk(bufB, base + (sB + j * 4 + u) * L)
                return c2

            def b1(j, c2, base=base, sB=sB):
                blk(bufB, base + (sB + j) * L)
                return c2

            lax.fori_loop(0, kB4, b4, 0)
            lax.fori_loop(kB4 * 4, kB, b1, 0)

            if h == 1:
                @pl.when(i + 1 < ROWS_PER_W)
                def _():
                    rn = r + 1
                    pltpu.async_copy(
                        tab_hbm.at[rn >> 6, rn & (D_TOKEN - 1),
                                   pl.ds(TSPLIT, VOCAB - TSPLIT)], bufB,
                        semB)

            pltpu.sync_copy(out_v, out_hbm.at[f, d, pl.ds(base, HB)])

        return f

    lax.fori_loop(0, ROWS_PER_W, row_step, -1)


@jax.jit
def _lookup(xc_t, tab_t):
    mesh = plsc.VectorSubcoreMesh(core_axis_name="c", subcore_axis_name="s")
    return pl.kernel(
        _body,
        mesh=mesh,
        out_type=jax.ShapeDtypeStruct((N_FIELDS, D_TOKEN, BATCH),
                                      jnp.float32),
        scratch_types=[
            pltpu.VMEM((TSPLIT,), jnp.float32),
            pltpu.VMEM((VOCAB - TSPLIT,), jnp.float32),
            pltpu.VMEM((BATCH,), jnp.int32),
            pltpu.VMEM((HB,), jnp.float32),
            pltpu.VMEM((SC_CHUNK,), jnp.int32),
            pltpu.SMEM((8,), jnp.int32),
            pltpu.SemaphoreType.DMA,
            pltpu.SemaphoreType.DMA,
        ],
        compiler_params=pltpu.CompilerParams(needs_layout_passes=False),
    )(xc_t, tab_t)


def kernel(x_cat, tables):
    xc_t = x_cat.astype(jnp.int32).T
    tab_t = jnp.transpose(tables, (0, 2, 1))
    out_t = _lookup(xc_t, tab_t)
    return jnp.transpose(out_t, (2, 0, 1))
